# baseline (speedup 1.0000x reference)
import functools

import jax
import jax.numpy as jnp
import numpy as np
from jax import lax
from jax.experimental import pallas as pl
from jax.experimental.pallas import tpu as pltpu
from jax.experimental.pallas import tpu_sc as plsc

_K = 8
_CARD = 2051
_D = 128
_B = 16
_T = 2048
_NC = 2
_NS = 16
_NW = _NC * _NS
_ROWS = _B * _T
_RPW = _ROWS // _NW
_CHUNK = 16
_GROWS = _K * _CHUNK
_NCHUNK = _RPW // _CHUNK
_LANES = 16

_PERM = np.concatenate([
    g * 32 + np.stack([np.arange(16), np.arange(16) + 16], axis=1).reshape(32)
    for g in range(_D // 32)
])


def _body(emb_hbm, idx_hbm, out_hbm, idxraw, idx2, gbuf, obuf,
          gsem0, gsem1, osem0, osem1):
    wid = lax.axis_index("c") * _NS + lax.axis_index("s")
    b = wid // 2
    half = wid % 2
    base = wid * _RPW

    for k in range(_K):
        pltpu.sync_copy(idx_hbm.at[b * _K + k, pl.ds(half * _RPW, _RPW)],
                        idxraw.at[k])

    def build_idx(c, carry):
        for k in range(_K):
            idx2[c, pl.ds(k * _LANES, _LANES)] = (
                idxraw[k, pl.ds(c * _CHUNK, _CHUNK)] + k * _CARD)
        return carry
    lax.fori_loop(0, _NCHUNK, build_idx, 0)

    gsems = (gsem0, gsem1)
    osems = (osem0, osem1)

    def fire_gather(c, s):
        pltpu.async_copy(emb_hbm.at[idx2.at[c]], gbuf.at[s], gsems[s])

    def drain_gather(s):
        pltpu.make_async_copy(emb_hbm.at[pl.ds(0, _GROWS)], gbuf.at[s],
                              gsems[s]).wait()

    def drain_out(s):
        pltpu.make_async_copy(obuf.at[s], out_hbm.at[pl.ds(base, _CHUNK)],
                              osems[s]).wait()

    for s in range(2):
        fire_gather(s, s)

    himask = jnp.int32(-65536)

    def outer(g, carry):
        for s in range(2):
            c = g * 2 + s
            drain_gather(s)

            @pl.when(c >= 2)
            def _():
                drain_out(s)

            def sum_rows(r, rc):
                for col in range(_D // 32):
                    ds_ = pl.ds(col * 16, _LANES)
                    ws = [gbuf[s, k * _CHUNK + r, ds_] for k in range(_K)]
                    los = [lax.bitcast_convert_type(lax.shift_left(w, 16),
                                                    jnp.float32) for w in ws]
                    his = [lax.bitcast_convert_type(w & himask, jnp.float32)
                           for w in ws]
                    lo = ((los[0] + los[1]) + (los[2] + los[3])) + \
                         ((los[4] + los[5]) + (los[6] + los[7]))
                    hi = ((his[0] + his[1]) + (his[2] + his[3])) + \
                         ((his[4] + his[5]) + (his[6] + his[7]))
                    obuf[s, r, pl.ds(col * 32, _LANES)] = lo
                    obuf[s, r, pl.ds(col * 32 + _LANES, _LANES)] = hi
                return rc
            lax.fori_loop(0, _CHUNK, sum_rows, 0)

            pltpu.async_copy(obuf.at[s],
                             out_hbm.at[pl.ds(base + c * _CHUNK, _CHUNK)],
                             osems[s])

            @pl.when(c + 2 < _NCHUNK)
            def _():
                fire_gather(c + 2, s)
        return carry
    lax.fori_loop(0, _NCHUNK // 2, outer, 0)

    drain_out(0)
    drain_out(1)


@jax.jit
def _embed_sum(emb2d, idx2d):
    mesh = plsc.VectorSubcoreMesh(core_axis_name="c", subcore_axis_name="s")
    kfn = pl.kernel(
        _body,
        out_type=jax.ShapeDtypeStruct((_ROWS, _D), jnp.float32),
        mesh=mesh,
        compiler_params=pltpu.CompilerParams(use_tc_tiling_on_sc=False),
        scratch_types=[
            pltpu.VMEM((_K, _RPW), jnp.int32),
            pltpu.VMEM((_NCHUNK, _GROWS), jnp.int32),
            pltpu.VMEM((2, _GROWS, _D // 2), jnp.int32),
            pltpu.VMEM((2, _CHUNK, _D), jnp.float32),
            pltpu.SemaphoreType.DMA,
            pltpu.SemaphoreType.DMA,
            pltpu.SemaphoreType.DMA,
            pltpu.SemaphoreType.DMA,
        ],
    )
    return kfn(emb2d, idx2d)


def kernel(indices, emb):
    idx2d = indices.reshape(_B * _K, _T).astype(jnp.int32)
    emb_bf = emb.reshape(_K * _CARD, _D)[:, _PERM].astype(jnp.bfloat16)
    emb_pk = jax.lax.bitcast_convert_type(
        emb_bf.reshape(_K * _CARD, _D // 2, 2), jnp.int32)
    out = _embed_sum(emb_pk, idx2d)
    return out.reshape(_B, _T, _D)
